# baseline (speedup 1.0000x reference)
import jax
import jax.numpy as jnp
from jax.experimental import pallas as pl

_K = 1000
_I1 = 6
_I2 = 16


def _mask_kernel(data_ref, wt_ref, b_ref, out_ref):
    x = data_ref[...]
    w = wt_ref[...]
    out = jnp.dot(x, w, preferred_element_type=jnp.float32) + b_ref[...]

    rmax = jnp.max(out, axis=1, keepdims=True)
    rmin = jnp.min(out, axis=1, keepdims=True)
    span = rmax - rmin
    lo = rmin - (span * 1e-3 + 1e-30)
    hi = rmax
    for _ in range(_I1):
        mid = 0.5 * (lo + hi)
        cnt = jnp.sum((out > mid).astype(jnp.float32), axis=1, keepdims=True)
        pred = cnt >= _K
        lo = jnp.where(pred, mid, lo)
        hi = jnp.where(pred, hi, mid)

    scale = 65534.0 / (hi - lo)
    zf = jnp.clip((out - lo) * scale, 0.0, 65534.0)
    z = (zf - 32767.0).astype(jnp.int16)

    ones_col = jnp.ones((z.shape[1], 1), jnp.bfloat16)
    mlo = jnp.full(lo.shape, -32768, jnp.int32)
    mhi = jnp.full(lo.shape, 32767, jnp.int32)
    for _ in range(_I2):
        mmid = (mlo + mhi) >> 1
        gt = jnp.where(z > mmid.astype(jnp.int16),
                       jnp.bfloat16(1.0), jnp.bfloat16(0.0))
        cnt = jnp.dot(gt, ones_col, preferred_element_type=jnp.float32)
        pred = cnt >= _K
        mlo = jnp.where(pred, mmid, mlo)
        mhi = jnp.where(pred, mhi, mmid)
    out_ref[...] = jnp.where(z > mlo.astype(jnp.int16), out, 0.0)


def kernel(data, W, b):
    B, D = data.shape
    V = W.shape[0]
    R = 32 if B % 32 == 0 else (8 if B % 8 == 0 else B)
    wt = W.T
    b2 = b.reshape(1, V)
    return pl.pallas_call(
        _mask_kernel,
        grid=(B // R,),
        in_specs=[
            pl.BlockSpec((R, D), lambda i: (i, 0)),
            pl.BlockSpec((D, V), lambda i: (0, 0)),
            pl.BlockSpec((1, V), lambda i: (0, 0)),
        ],
        out_specs=pl.BlockSpec((R, V), lambda i: (i, 0)),
        out_shape=jax.ShapeDtypeStruct((B, V), jnp.float32),
    )(data, wt, b2)


# speedup vs baseline: 2.4235x; 2.4235x over previous
import jax
import jax.numpy as jnp
from jax.experimental import pallas as pl

_K = 1000
_N_BISECT = 19


def _mask_kernel(data_ref, wt_ref, b_ref, out_ref):
    x = data_ref[...]
    w = wt_ref[...]
    out = jnp.dot(x, w, preferred_element_type=jnp.float32) + b_ref[...]

    rmax = jnp.max(out, axis=1, keepdims=True)
    rmin = jnp.min(out, axis=1, keepdims=True)
    span = rmax - rmin
    lo0 = rmin - (span * 1e-3 + 1e-30)
    hi0 = rmax

    lo, hi = lo0, hi0
    for _ in range(_N_BISECT):
        mid = 0.5 * (lo + hi)
        cnt = jnp.sum((out > mid).astype(jnp.float32), axis=1, keepdims=True)
        pred = cnt >= _K
        lo = jnp.where(pred, mid, lo)
        hi = jnp.where(pred, hi, mid)
    out_ref[...] = jnp.where(out > lo, out, 0.0)


def kernel(data, W, b):
    B, D = data.shape
    V = W.shape[0]
    R = 32 if B % 32 == 0 else (8 if B % 8 == 0 else B)
    wt = W.T
    b2 = b.reshape(1, V)
    return pl.pallas_call(
        _mask_kernel,
        grid=(B // R,),
        in_specs=[
            pl.BlockSpec((R, D), lambda i: (i, 0)),
            pl.BlockSpec((D, V), lambda i: (0, 0)),
            pl.BlockSpec((1, V), lambda i: (0, 0)),
        ],
        out_specs=pl.BlockSpec((R, V), lambda i: (i, 0)),
        out_shape=jax.ShapeDtypeStruct((B, V), jnp.float32),
    )(data, wt, b2)
